# baseline (speedup 1.0000x reference)
import jax
import jax.numpy as jnp
from jax import lax
from jax.experimental import pallas as pl
from jax.experimental.pallas import tpu as pltpu


def _edge_attn_body(idx3_hbm, src_hbm, w1t_hbm, prelu_hbm, w2t_hbm, out_ref,
                    idx3_vmem, src_vmem, w1t_vmem, w2t_vmem, prelu_smem,
                    sem_i, sem_s, sem_w1, sem_w2, sem_p):
    F, L, F2 = idx3_vmem.shape
    P = F * F2
    B, N, C = src_vmem.shape
    HID = w1t_vmem.shape[0]

    idx_copy = pltpu.make_async_copy(idx3_hbm, idx3_vmem, sem_i)
    idx_copy.start()
    src_copy = pltpu.make_async_copy(src_hbm, src_vmem, sem_s)
    src_copy.start()
    w1_copy = pltpu.make_async_copy(w1t_hbm, w1t_vmem, sem_w1)
    w1_copy.start()
    w2_copy = pltpu.make_async_copy(w2t_hbm, w2t_vmem, sem_w2)
    w2_copy.start()
    p_copy = pltpu.make_async_copy(prelu_hbm, prelu_smem, sem_p)
    p_copy.start()

    sp0 = lax.broadcasted_iota(jnp.int32, (F2, P), 0)
    sp1 = lax.broadcasted_iota(jnp.int32, (F2, P), 1)
    spread = ((sp1 % F2) == sp0).astype(jnp.float32)
    ms0 = lax.broadcasted_iota(jnp.int32, (F * L, P), 0)
    ms1 = lax.broadcasted_iota(jnp.int32, (F * L, P), 1)
    msel = ((ms1 // F2) == (ms0 // L)).astype(jnp.float32)
    e80 = lax.broadcasted_iota(jnp.int32, (L, F * L), 0)
    e81 = lax.broadcasted_iota(jnp.int32, (L, F * L), 1)
    e8 = ((e81 % L) == e80).astype(jnp.float32)

    node_iota = lax.broadcasted_iota(jnp.int32, (P, N), 1)

    idx_copy.wait()
    idx_flat = idx3_vmem[...].astype(jnp.float32).reshape(F * L, F2)
    yk = lax.dot(idx_flat, spread,
                 preferred_element_type=jnp.float32)
    idx_cols = lax.dot_general(yk * msel, e8,
                               (((0,), (1,)), ((), ())),
                               preferred_element_type=jnp.float32)

    idx_cols_i = idx_cols.astype(jnp.int32)
    counts = jnp.zeros((P, N), jnp.float32)
    for k in range(L - 1):
        counts += (idx_cols_i[:, k:k + 1] == node_iota).astype(jnp.float32)

    dcounts = counts - counts

    w1_copy.wait()
    w2_copy.wait()
    p_copy.wait()
    w1t = w1t_vmem[...]
    w2t = w2t_vmem[...]
    p_neg = prelu_smem[0]

    rs0 = lax.broadcasted_iota(jnp.int32, (2 * F, P), 0)
    rs1 = lax.broadcasted_iota(jnp.int32, (2 * F, P), 1)
    rsel = ((rs1 // F2) * 2 == rs0).astype(jnp.float32)
    cs0 = lax.broadcasted_iota(jnp.int32, (P, 128), 0)
    cs1 = lax.broadcasted_iota(jnp.int32, (P, 128), 1)
    csel = ((cs0 % F2) == cs1).astype(jnp.float32)

    src_copy.wait()

    dcb = dcounts.astype(jnp.bfloat16)
    src_all = jnp.concatenate(
        [src_vmem[b].astype(jnp.bfloat16) for b in range(B)], axis=1)
    contrib_all = lax.dot(dcb, src_all,
                          preferred_element_type=jnp.float32)

    out_ref[...] = jnp.zeros(out_ref.shape, jnp.float32)
    rows_per_batch = 2 * N
    for b in range(B):
        contrib = contrib_all[:, b * C:(b + 1) * C]
        h = lax.dot_general(contrib, w1t,
                            (((1,), (1,)), ((), ())),
                            preferred_element_type=jnp.float32)
        h = jnp.where(h >= 0, h, p_neg * h)
        hwT = lax.dot_general(w2t, h,
                              (((1,), (1,)), ((), ())),
                              preferred_element_type=jnp.float32)
        att_rows = lax.dot(rsel * hwT, csel,
                           preferred_element_type=jnp.float32)
        out_ref[pl.ds(b * rows_per_batch, 2 * F), :] = att_rows


def kernel(src, t_SPD, W1, prelu_w, W2):
    B, N, C = src.shape
    F = t_SPD.shape[0]
    L = t_SPD.shape[2]
    HID = W1.shape[1]

    idx3 = t_SPD.transpose(0, 2, 1)
    out = pl.pallas_call(
        _edge_attn_body,
        out_shape=jax.ShapeDtypeStruct((B * N * (N // 128), 128), jnp.float32),
        in_specs=[
            pl.BlockSpec(memory_space=pl.ANY),
            pl.BlockSpec(memory_space=pl.ANY),
            pl.BlockSpec(memory_space=pl.ANY),
            pl.BlockSpec(memory_space=pl.ANY),
            pl.BlockSpec(memory_space=pl.ANY),
        ],
        scratch_shapes=[
            pltpu.VMEM((F, L, F), jnp.int32),
            pltpu.VMEM((B, N, C), jnp.float32),
            pltpu.VMEM((HID, C), jnp.float32),
            pltpu.VMEM((1, HID), jnp.float32),
            pltpu.SMEM((1,), jnp.float32),
            pltpu.SemaphoreType.DMA,
            pltpu.SemaphoreType.DMA,
            pltpu.SemaphoreType.DMA,
            pltpu.SemaphoreType.DMA,
            pltpu.SemaphoreType.DMA,
        ],
    )(idx3, src, W1.T, prelu_w, W2.reshape(1, HID))
    return out.reshape(B, N, N, 1)


# speedup vs baseline: 1.0617x; 1.0617x over previous
import jax
import jax.numpy as jnp
from jax import lax
from jax.experimental import pallas as pl
from jax.experimental.pallas import tpu as pltpu


def _edge_attn_body(idx3_hbm, src_hbm, w1t_hbm, prelu_hbm, w2t_hbm, out_hbm,
                    idx3_vmem, src_vmem, w1t_vmem, w2t_vmem, prelu_smem,
                    zero_vmem, att_vmem,
                    sem_i, sem_s, sem_w1, sem_w2, sem_p, sem_z, sem_a):
    F, L, F2 = idx3_vmem.shape
    P = F * F2
    B, N, C = src_vmem.shape
    HID = w1t_vmem.shape[0]

    idx_copy = pltpu.make_async_copy(idx3_hbm, idx3_vmem, sem_i)
    idx_copy.start()
    src_copy = pltpu.make_async_copy(src_hbm, src_vmem, sem_s)
    src_copy.start()
    w1_copy = pltpu.make_async_copy(w1t_hbm, w1t_vmem, sem_w1)
    w1_copy.start()
    w2_copy = pltpu.make_async_copy(w2t_hbm, w2t_vmem, sem_w2)
    w2_copy.start()
    p_copy = pltpu.make_async_copy(prelu_hbm, prelu_smem, sem_p)
    p_copy.start()

    zero_vmem[...] = jnp.zeros(zero_vmem.shape, jnp.float32)
    z_copy = pltpu.make_async_copy(zero_vmem, out_hbm, sem_z)
    z_copy.start()

    sp0 = lax.broadcasted_iota(jnp.int32, (F2, P), 0)
    sp1 = lax.broadcasted_iota(jnp.int32, (F2, P), 1)
    spread = ((sp1 % F2) == sp0).astype(jnp.float32)
    ms0 = lax.broadcasted_iota(jnp.int32, (F * L, P), 0)
    ms1 = lax.broadcasted_iota(jnp.int32, (F * L, P), 1)
    msel = ((ms1 // F2) == (ms0 // L)).astype(jnp.float32)
    e80 = lax.broadcasted_iota(jnp.int32, (L, F * L), 0)
    e81 = lax.broadcasted_iota(jnp.int32, (L, F * L), 1)
    e8 = ((e81 % L) == e80).astype(jnp.float32)

    node_iota = lax.broadcasted_iota(jnp.int32, (P, N), 1)

    idx_copy.wait()
    idx_flat = idx3_vmem[...].astype(jnp.float32).reshape(F * L, F2)
    yk = lax.dot(idx_flat, spread,
                 preferred_element_type=jnp.float32)
    idx_cols = lax.dot_general(yk * msel, e8,
                               (((0,), (1,)), ((), ())),
                               preferred_element_type=jnp.float32)

    idx_cols_i = idx_cols.astype(jnp.int32)
    counts = jnp.zeros((P, N), jnp.float32)
    for k in range(L - 1):
        counts += (idx_cols_i[:, k:k + 1] == node_iota).astype(jnp.float32)

    dcounts = counts - counts

    w1_copy.wait()
    w2_copy.wait()
    p_copy.wait()
    w1t = w1t_vmem[...]
    w2t = w2t_vmem[...]
    p_neg = prelu_smem[0]

    rs0 = lax.broadcasted_iota(jnp.int32, (2 * F, P), 0)
    rs1 = lax.broadcasted_iota(jnp.int32, (2 * F, P), 1)
    rsel = ((rs1 // F2) * 2 == rs0).astype(jnp.float32)
    cs0 = lax.broadcasted_iota(jnp.int32, (P, 128), 0)
    cs1 = lax.broadcasted_iota(jnp.int32, (P, 128), 1)
    csel = ((cs0 % F2) == cs1).astype(jnp.float32)

    src_copy.wait()

    dcb = dcounts.astype(jnp.bfloat16)
    src_all = jnp.concatenate(
        [src_vmem[b].astype(jnp.bfloat16) for b in range(B)], axis=1)
    contrib_all = lax.dot(dcb, src_all,
                          preferred_element_type=jnp.float32)

    rows_per_batch = 2 * N
    for b in range(B):
        contrib = contrib_all[:, b * C:(b + 1) * C]
        h = lax.dot_general(contrib, w1t,
                            (((1,), (1,)), ((), ())),
                            preferred_element_type=jnp.float32)
        h = jnp.where(h >= 0, h, p_neg * h)
        hwT = lax.dot_general(w2t, h,
                              (((1,), (1,)), ((), ())),
                              preferred_element_type=jnp.float32)
        att_rows = lax.dot(rsel * hwT, csel,
                           preferred_element_type=jnp.float32)
        att_vmem[pl.ds(b * 2 * F, 2 * F), :] = att_rows

    z_copy.wait()
    corner_copies = []
    for b in range(B):
        c = pltpu.make_async_copy(
            att_vmem.at[pl.ds(b * 2 * F, 2 * F)],
            out_hbm.at[pl.ds(b * rows_per_batch, 2 * F)], sem_a)
        c.start()
        corner_copies.append(c)
    for c in corner_copies:
        c.wait()


def kernel(src, t_SPD, W1, prelu_w, W2):
    B, N, C = src.shape
    F = t_SPD.shape[0]
    L = t_SPD.shape[2]
    HID = W1.shape[1]

    idx3 = t_SPD.transpose(0, 2, 1)
    out = pl.pallas_call(
        _edge_attn_body,
        out_shape=jax.ShapeDtypeStruct((B * N * (N // 128), 128), jnp.float32),
        out_specs=pl.BlockSpec(memory_space=pl.ANY),
        in_specs=[
            pl.BlockSpec(memory_space=pl.ANY),
            pl.BlockSpec(memory_space=pl.ANY),
            pl.BlockSpec(memory_space=pl.ANY),
            pl.BlockSpec(memory_space=pl.ANY),
            pl.BlockSpec(memory_space=pl.ANY),
        ],
        scratch_shapes=[
            pltpu.VMEM((F, L, F), jnp.int32),
            pltpu.VMEM((B, N, C), jnp.float32),
            pltpu.VMEM((HID, C), jnp.float32),
            pltpu.VMEM((1, HID), jnp.float32),
            pltpu.SMEM((1,), jnp.float32),
            pltpu.VMEM((B * N * (N // 128), 128), jnp.float32),
            pltpu.VMEM((B * 2 * F, 128), jnp.float32),
            pltpu.SemaphoreType.DMA,
            pltpu.SemaphoreType.DMA,
            pltpu.SemaphoreType.DMA,
            pltpu.SemaphoreType.DMA,
            pltpu.SemaphoreType.DMA,
            pltpu.SemaphoreType.DMA,
            pltpu.SemaphoreType.DMA,
        ],
    )(idx3, src, W1.T, prelu_w, W2.reshape(1, HID))
    return out.reshape(B, N, N, 1)
